# baseline (speedup 1.0000x reference)
import functools

import jax
import jax.numpy as jnp
from jax import lax
from jax.experimental import pallas as pl
from jax.experimental.pallas import tpu as pltpu
from jax.experimental.pallas import tpu_sc as plsc

N = 10000
E = 320000
D = 128
DE = 16
H = 128
C = 32
G = 128

NC = 2
NS = 16
NW = NC * NS
E_PAD = 327680
E_PER_W = E_PAD // NW
CHUNK = 16
NCHUNK = E_PER_W // CHUNK
NGRP = 5
GCH = NCHUNK // NGRP
NQUAD = GCH // 4
NA = 10240
ROWS_PER_SUB = NA // NS


def _sc_layer_body(h_hbm, src_hbm, dst_hbm, ep_hbm, zero_hbm, out_hbm,
                   src_v, dst_v, rows0, rows1, rows2, rows3,
                   ep0, ep1, ep2, ep3, acc_sh,
                   g0, g1, g2, g3, e0, e1, e2, e3, s0, s1, s2, s3):
    cid = lax.axis_index("c")
    sid = lax.axis_index("s")
    wid = sid * NC + cid

    rows = (rows0, rows1, rows2, rows3)
    eps = (ep0, ep1, ep2, ep3)
    gsem = (g0, g1, g2, g3)
    esem = (e0, e1, e2, e3)
    ssem = (s0, s1, s2, s3)

    pltpu.sync_copy(zero_hbm.at[pl.ds(sid * ROWS_PER_SUB, ROWS_PER_SUB)],
                    acc_sh.at[pl.ds(sid * ROWS_PER_SUB, ROWS_PER_SUB)])
    plsc.subcore_barrier()

    ep_base = wid * E_PER_W

    def compute(p):
        rows_v, ep_v = rows[p], eps[p]

        @plsc.parallel_loop(0, CHUNK, unroll=4)
        def row_body(r):
            for j in range(D // 16):
                s = pl.ds(j * 16, 16)
                v = rows_v[r, s] + ep_v[r, s]
                rows_v[r, s] = jnp.maximum(v, 0.0)

    for grp in range(NGRP):
        goff = grp * GCH

        def issue(c_grp, p, goff=goff):
            pltpu.async_copy(h_hbm.at[src_v.at[c_grp]], rows[p], gsem[p])
            pltpu.async_copy(
                ep_hbm.at[pl.ds(ep_base + (goff + c_grp) * CHUNK, CHUNK)],
                eps[p], esem[p])

        def wait_gather(c_grp, p):
            pltpu.make_async_copy(
                h_hbm.at[src_v.at[c_grp]], rows[p], gsem[p]).wait()
            pltpu.make_async_copy(
                ep_hbm.at[pl.ds(ep_base, CHUNK)], eps[p], esem[p]).wait()

        def scatter(c_grp, p):
            pltpu.async_copy(rows[p], acc_sh.at[dst_v.at[c_grp]], ssem[p],
                             add=True)

        def wait_scatter(p):
            pltpu.make_async_copy(rows[p], acc_sh.at[dst_v.at[0]],
                                  ssem[p]).wait()

        pltpu.sync_copy(src_hbm.at[wid, grp], src_v)
        pltpu.sync_copy(dst_hbm.at[wid, grp], dst_v)
        issue(0, 0)
        issue(1, 1)

        def quad_body(k, carry):
            for p in range(4):
                c = 4 * k + p
                pfree = (p + 2) % 4
                if p < 2:
                    @pl.when(k > 0)
                    def _():
                        wait_scatter(pfree)

                    issue(c + 2, pfree)
                else:
                    wait_scatter(pfree)

                    @pl.when(k < NQUAD - 1)
                    def _():
                        issue(c + 2, pfree)

                wait_gather(c, p)
                compute(p)
                scatter(c, p)
            return carry

        lax.fori_loop(0, NQUAD, quad_body, 0)
        wait_scatter(2)
        wait_scatter(3)

    plsc.subcore_barrier()
    pltpu.sync_copy(acc_sh.at[pl.ds(sid * ROWS_PER_SUB, ROWS_PER_SUB)],
                    out_hbm.at[cid, pl.ds(sid * ROWS_PER_SUB, ROWS_PER_SUB)])


_sc_layer = pl.kernel(
    _sc_layer_body,
    out_type=jax.ShapeDtypeStruct((NC, NA, D), jnp.float32),
    mesh=plsc.VectorSubcoreMesh(core_axis_name="c", subcore_axis_name="s"),
    scratch_types=(
        [pltpu.VMEM((GCH, CHUNK), jnp.int32)] * 2
        + [pltpu.VMEM((CHUNK, D), jnp.float32)] * 8
        + [pltpu.VMEM_SHARED((NA, D), jnp.float32)]
        + [pltpu.SemaphoreType.DMA] * 12
    ),
)


def _eproj_body(ea_ref, w_ref, b_ref, o_ref):
    o_ref[...] = jnp.dot(ea_ref[...], w_ref[...],
                         preferred_element_type=jnp.float32) + b_ref[...]


_EBLK = 2048


def _eproj(edge_attr, w, b):
    return pl.pallas_call(
        _eproj_body,
        grid=(E_PAD // _EBLK,),
        in_specs=[
            pl.BlockSpec((_EBLK, DE), lambda i: (i, 0)),
            pl.BlockSpec((DE, D), lambda i: (0, 0)),
            pl.BlockSpec((1, D), lambda i: (0, 0)),
        ],
        out_specs=pl.BlockSpec((_EBLK, D), lambda i: (i, 0)),
        out_shape=jax.ShapeDtypeStruct((E_PAD, D), jnp.float32),
    )(edge_attr, w, b)


def _update_body(h_ref, agg_ref, w_ref, b_ref, o_ref):
    s = h_ref[...] + agg_ref[0] + agg_ref[1]
    m = jnp.dot(s, w_ref[...], preferred_element_type=jnp.float32) + b_ref[...]
    o_ref[...] = jnp.where(m > 0, m, 0.01 * m)


_NBLK = 1000


def _update(h, aggs, w, b):
    return pl.pallas_call(
        _update_body,
        grid=(N // _NBLK,),
        in_specs=[
            pl.BlockSpec((_NBLK, D), lambda i: (i, 0)),
            pl.BlockSpec((NC, _NBLK, D), lambda i: (0, i, 0)),
            pl.BlockSpec((D, H), lambda i: (0, 0)),
            pl.BlockSpec((1, H), lambda i: (0, 0)),
        ],
        out_specs=pl.BlockSpec((_NBLK, H), lambda i: (i, 0)),
        out_shape=jax.ShapeDtypeStruct((N, H), jnp.float32),
    )(h, aggs, w, b)


def _pool_head_body(h_ref, batch_ref, l0w_ref, l0b_ref, l1w_ref, l1b_ref,
                    o_ref, sums_ref, cnts_ref):
    i = pl.program_id(0)

    @pl.when(i == 0)
    def _():
        sums_ref[...] = jnp.zeros_like(sums_ref)
        cnts_ref[...] = jnp.zeros_like(cnts_ref)

    gids = lax.broadcasted_iota(jnp.int32, (G, _PBLK), 0)
    mask = (gids == batch_ref[0]).astype(jnp.float32)
    sums_ref[...] += jnp.dot(mask, h_ref[...],
                             preferred_element_type=jnp.float32)
    cnts_ref[...] += jnp.dot(mask, jnp.ones((_PBLK, H), jnp.float32),
                             preferred_element_type=jnp.float32)

    @pl.when(i == pl.num_programs(0) - 1)
    def _():
        pooled = sums_ref[...] / jnp.maximum(cnts_ref[...], 1.0)
        z = jnp.dot(pooled, l0w_ref[...],
                    preferred_element_type=jnp.float32) + l0b_ref[...]
        z = jnp.where(z > 0, z, 0.01 * z)
        z = jnp.dot(z, l1w_ref[...],
                    preferred_element_type=jnp.float32) + l1b_ref[...]
        m = jnp.max(z, axis=1, keepdims=True)
        zs = z - m
        lse = jnp.log(jnp.sum(jnp.exp(zs), axis=1, keepdims=True))
        o_ref[...] = zs - lse


_PBLK = 1000


def _pool_head(h, batch2d, l0w, l0b, l1w, l1b):
    return pl.pallas_call(
        _pool_head_body,
        grid=(N // _PBLK,),
        in_specs=[
            pl.BlockSpec((_PBLK, H), lambda i: (i, 0)),
            pl.BlockSpec((1, 1, _PBLK), lambda i: (i, 0, 0)),
            pl.BlockSpec((H, H), lambda i: (0, 0)),
            pl.BlockSpec((1, H), lambda i: (0, 0)),
            pl.BlockSpec((H, C), lambda i: (0, 0)),
            pl.BlockSpec((1, C), lambda i: (0, 0)),
        ],
        out_specs=pl.BlockSpec((G, C), lambda i: (0, 0)),
        out_shape=jax.ShapeDtypeStruct((G, C), jnp.float32),
        scratch_shapes=[
            pltpu.VMEM((G, H), jnp.float32),
            pltpu.VMEM((G, H), jnp.float32),
        ],
    )(h, batch2d, l0w, l0b, l1w, l1b)


def kernel(x, edge_index, edge_attr, batch,
           We0, be0, W0, b0,
           We1, be1, W1, b1,
           We2, be2, W2, b2,
           L0W, L0b, L1W, L1b):
    npad = E_PAD - E
    src = jnp.concatenate(
        [edge_index[0].astype(jnp.int32),
         jnp.arange(npad, dtype=jnp.int32) % N])
    dst = jnp.concatenate(
        [edge_index[1].astype(jnp.int32),
         N + jnp.arange(npad, dtype=jnp.int32) % (NA - N)])
    src = src.reshape(NW, NGRP, GCH, CHUNK)
    dst = dst.reshape(NW, NGRP, GCH, CHUNK)
    eap = jnp.concatenate([edge_attr, jnp.zeros((npad, DE), jnp.float32)])
    zeros_n = jnp.zeros((NA, D), jnp.float32)

    h = x
    for we, be, w, b in ((We0, be0, W0, b0), (We1, be1, W1, b1),
                         (We2, be2, W2, b2)):
        ep = _eproj(eap, we, be.reshape(1, D))
        aggs = _sc_layer(h, src, dst, ep, zeros_n)
        h = _update(h, aggs, w, b.reshape(1, H))

    out = _pool_head(h, batch.astype(jnp.int32).reshape(N // _PBLK, 1, _PBLK),
                     L0W, L0b.reshape(1, H), L1W, L1b.reshape(1, C))
    return out
